# baseline (speedup 1.0000x reference)
import jax
import jax.numpy as jnp
from jax import lax
from jax.experimental import pallas as pl
from jax.experimental.pallas import tpu as pltpu
from jax.experimental.pallas import tpu_sc as plsc

B, L, D, V = 4096, 200, 128, 100000
NC, NS, LANES = 2, 16, 16
NW = NC * NS
BPW = B // NW
NCHUNK = D // LANES
NBUF = 4


def _emb_body(xt_hbm, table_hbm, pe_hbm, out_hbm, idx_v, pe_v, bufs, semg, sems):
    wid = lax.axis_index("s") * NC + lax.axis_index("c")
    b0 = wid * BPW

    pltpu.sync_copy(xt_hbm.at[:, pl.ds(b0, BPW)], idx_v)
    pltpu.sync_copy(pe_hbm, pe_v)

    def gather_fire(l, j):
        pltpu.async_copy(table_hbm.at[idx_v.at[l]], bufs[j], semg[j])

    def gather_wait(l, j):
        pltpu.make_async_copy(table_hbm.at[idx_v.at[l]], bufs[j], semg[j]).wait()

    def store_fire(l, j):
        pltpu.async_copy(bufs[j], out_hbm.at[pl.ds(b0, BPW), l], sems[j])

    def store_wait(l, j):
        pltpu.make_async_copy(bufs[j], out_hbm.at[pl.ds(b0, BPW), l], sems[j]).wait()

    def add_pe(l, j):
        buf = bufs[j]
        pevs = [pe_v[l, pl.ds(c * LANES, LANES)] for c in range(NCHUNK)]

        @plsc.parallel_loop(0, BPW, 1, unroll=8)
        def _body(b):
            for c in range(NCHUNK):
                sl = pl.ds(c * LANES, LANES)
                buf[b, sl] = buf[b, sl] + pevs[c]

    def step(l, jj, first, last):
        gather_wait(l, jj)
        add_pe(l, jj)
        store_fire(l, jj)
        j2 = (jj + 2) % NBUF
        if not first:
            store_wait(l - 2, j2)
        if not last:
            gather_fire(l + 2, j2)

    gather_fire(0, 0)
    gather_fire(1, 1)

    step(0, 0, True, False)
    step(1, 1, True, False)

    def outer(i, carry):
        base = 2 + i * NBUF
        for jj in range(NBUF):
            l = base + jj
            step(l, (2 + jj) % NBUF, False, False)
        return carry

    lax.fori_loop(0, (L - 4) // NBUF, outer, 0)

    step(L - 2, (L - 2) % NBUF, False, True)
    step(L - 1, (L - 1) % NBUF, False, True)
    store_wait(L - 2, (L - 2) % NBUF)
    store_wait(L - 1, (L - 1) % NBUF)


def kernel(x, table, pe):
    xt = x.T
    pe2 = pe.reshape(L, D)
    run = pl.kernel(
        _emb_body,
        out_type=jax.ShapeDtypeStruct((B, L, D), jnp.float32),
        mesh=plsc.VectorSubcoreMesh(core_axis_name="c", subcore_axis_name="s"),
        scratch_types=[
            pltpu.VMEM((L, BPW), jnp.int32),
            pltpu.VMEM((L, D), jnp.float32),
            [pltpu.VMEM((BPW, D), jnp.float32) for _ in range(NBUF)],
            [pltpu.SemaphoreType.DMA for _ in range(NBUF)],
            [pltpu.SemaphoreType.DMA for _ in range(NBUF)],
        ],
    )
    return run(xt, table, pe2)


# speedup vs baseline: 1.0033x; 1.0033x over previous
import jax
import jax.numpy as jnp
from jax import lax
from jax.experimental import pallas as pl
from jax.experimental.pallas import tpu as pltpu
from jax.experimental.pallas import tpu_sc as plsc

B, L, D, V = 4096, 200, 128, 100000
NC, NS, LANES = 2, 16, 16
NW = NC * NS
BPW = B // NW
NCHUNK = D // LANES
NBUF = 4


def _emb_body(xt_hbm, table_hbm, pe_hbm, out_hbm, idx_v, pe_v, bufs, semg, sems):
    wid = lax.axis_index("s") * NC + lax.axis_index("c")
    b0 = wid * BPW

    pltpu.sync_copy(xt_hbm.at[:, pl.ds(b0, BPW)], idx_v)
    pltpu.sync_copy(pe_hbm, pe_v)

    def gather_fire(l, j):
        pltpu.async_copy(table_hbm.at[idx_v.at[l]], bufs[j], semg[j])

    def gather_wait(l, j):
        pltpu.make_async_copy(table_hbm.at[idx_v.at[l]], bufs[j], semg[j]).wait()

    def store_fire(l, j):
        pltpu.async_copy(bufs[j], out_hbm.at[pl.ds(b0, BPW), l], sems[j])

    def store_wait(l, j):
        pltpu.make_async_copy(bufs[j], out_hbm.at[pl.ds(b0, BPW), l], sems[j]).wait()

    def add_pe(l, j):
        buf = bufs[j]
        pevs = [pe_v[l, pl.ds(c * LANES, LANES)] for c in range(NCHUNK)]

        @plsc.parallel_loop(0, BPW, 1, unroll=4)
        def _body(b):
            for c in range(NCHUNK):
                sl = pl.ds(c * LANES, LANES)
                buf[b, sl] = buf[b, sl] + pevs[c]

    def step(l, jj, first, last):
        gather_wait(l, jj)
        add_pe(l, jj)
        store_fire(l, jj)
        j2 = (jj + 2) % NBUF
        if not first:
            store_wait(l - 2, j2)
        if not last:
            gather_fire(l + 2, j2)

    gather_fire(0, 0)
    gather_fire(1, 1)

    step(0, 0, True, False)
    step(1, 1, True, False)

    def outer(i, carry):
        base = 2 + i * NBUF
        for jj in range(NBUF):
            l = base + jj
            step(l, (2 + jj) % NBUF, False, False)
        return carry

    lax.fori_loop(0, (L - 4) // NBUF, outer, 0)

    step(L - 2, (L - 2) % NBUF, False, True)
    step(L - 1, (L - 1) % NBUF, False, True)
    store_wait(L - 2, (L - 2) % NBUF)
    store_wait(L - 1, (L - 1) % NBUF)


def kernel(x, table, pe):
    xt = x.T
    pe2 = pe.reshape(L, D)
    run = pl.kernel(
        _emb_body,
        out_type=jax.ShapeDtypeStruct((B, L, D), jnp.float32),
        mesh=plsc.VectorSubcoreMesh(core_axis_name="c", subcore_axis_name="s"),
        scratch_types=[
            pltpu.VMEM((L, BPW), jnp.int32),
            pltpu.VMEM((L, D), jnp.float32),
            [pltpu.VMEM((BPW, D), jnp.float32) for _ in range(NBUF)],
            [pltpu.SemaphoreType.DMA for _ in range(NBUF)],
            [pltpu.SemaphoreType.DMA for _ in range(NBUF)],
        ],
    )
    return run(xt, table, pe2)
